# initial kernel scaffold (unmeasured)
import jax
import jax.numpy as jnp
from jax import lax
from jax.experimental import pallas as pl
from jax.experimental.pallas import tpu as pltpu

N_DEV = 32
LOG2_N = 5


def kernel(x, Wq, K_ext, V_ext, Wo):
    B, Sq, D = x.shape
    _, Skv, Hl, Dh = K_ext.shape
    Dq = Hl * Dh
    Dout = Wo.shape[1]

    def body(x_ref, wq_hbm, k_ref, v_ref, wo_hbm, out_ref,
             wq_s, wo_s, recv_ref, copy_sems, send_sems, recv_sems):
        my = lax.axis_index("i")

        cq = pltpu.make_async_copy(
            wq_hbm.at[:, pl.ds(my * Dq, Dq)], wq_s, copy_sems.at[0])
        co = pltpu.make_async_copy(
            wo_hbm.at[pl.ds(my * Dq, Dq), :], wo_s, copy_sems.at[1])
        cq.start()
        co.start()
        cq.wait()
        co.wait()

        for b in range(B):
            qb = jnp.dot(x_ref[b], wq_s[:, :],
                         preferred_element_type=jnp.float32)
            ctxs = []
            for h in range(Hl):
                q = qb[:, h * Dh:(h + 1) * Dh]
                k = k_ref[b, :, h, :]
                v = v_ref[b, :, h, :]
                s = lax.dot_general(
                    q, k, (((1,), (1,)), ((), ())),
                    preferred_element_type=jnp.float32) * 0.125
                m = jnp.max(s, axis=1, keepdims=True)
                w = jnp.exp(s - m)
                w = w / jnp.sum(w, axis=1, keepdims=True)
                ctxs.append(jnp.dot(w, v, preferred_element_type=jnp.float32))
            ctx = jnp.concatenate(ctxs, axis=1)
            out_ref[b] = jnp.dot(ctx, wo_s[:, :],
                                 preferred_element_type=jnp.float32)

        for st in range(LOG2_N):
            partner = my ^ (1 << st)
            rdma = pltpu.make_async_remote_copy(
                src_ref=out_ref,
                dst_ref=recv_ref.at[st],
                send_sem=send_sems.at[st],
                recv_sem=recv_sems.at[st],
                device_id=(partner,),
                device_id_type=pl.DeviceIdType.MESH,
            )
            rdma.start()
            rdma.wait()
            out_ref[:] = out_ref[:] + recv_ref[st]

    return pl.pallas_call(
        body,
        out_shape=jax.ShapeDtypeStruct((B, Sq, Dout), jnp.float32),
        in_specs=[
            pl.BlockSpec(memory_space=pltpu.VMEM),
            pl.BlockSpec(memory_space=pltpu.ANY),
            pl.BlockSpec(memory_space=pltpu.VMEM),
            pl.BlockSpec(memory_space=pltpu.VMEM),
            pl.BlockSpec(memory_space=pltpu.ANY),
        ],
        out_specs=pl.BlockSpec(memory_space=pltpu.VMEM),
        scratch_shapes=[
            pltpu.VMEM((D, Dq), jnp.float32),
            pltpu.VMEM((Dq, Dout), jnp.float32),
            pltpu.VMEM((LOG2_N, B, Sq, Dout), jnp.float32),
            pltpu.SemaphoreType.DMA((2,)),
            pltpu.SemaphoreType.DMA((LOG2_N,)),
            pltpu.SemaphoreType.DMA((LOG2_N,)),
        ],
        compiler_params=pltpu.CompilerParams(collective_id=0),
    )(x, Wq, K_ext, V_ext, Wo)


# baseline (device time: 76680 ns/iter reference)
import jax
import jax.numpy as jnp
from jax import lax
from jax.experimental import pallas as pl
from jax.experimental.pallas import tpu as pltpu

N_DEV = 32
LOG2_N = 5


def kernel(x, Wq, K_ext, V_ext, Wo):
    B, Sq, D = x.shape
    _, Skv, Hl, Dh = K_ext.shape
    Dq = Hl * Dh
    Dout = Wo.shape[1]

    def body(x_ref, wq_hbm, k_ref, v_ref, wo_hbm, out_ref,
             wq_s, wo_s, recv_ref, copy_sems, send_sems, recv_sems):
        my = lax.axis_index("i")

        cq = pltpu.make_async_copy(
            wq_hbm.at[:, pl.ds(my * Dq, Dq)], wq_s, copy_sems.at[0])
        co = pltpu.make_async_copy(
            wo_hbm.at[pl.ds(my * Dq, Dq), :], wo_s, copy_sems.at[1])
        cq.start()
        co.start()
        cq.wait()
        co.wait()

        for b in range(B):
            qb = jnp.dot(x_ref[b], wq_s[:, :],
                         preferred_element_type=jnp.float32)
            ctxs = []
            for h in range(Hl):
                q = qb[:, h * Dh:(h + 1) * Dh]
                k = k_ref[b, :, h, :]
                v = v_ref[b, :, h, :]
                s = lax.dot_general(
                    q, k, (((1,), (1,)), ((), ())),
                    preferred_element_type=jnp.float32) * 0.125
                m = jnp.max(s, axis=1, keepdims=True)
                w = jnp.exp(s - m)
                w = w / jnp.sum(w, axis=1, keepdims=True)
                ctxs.append(jnp.dot(w, v, preferred_element_type=jnp.float32))
            ctx = jnp.concatenate(ctxs, axis=1)
            out_ref[b] = jnp.dot(ctx, wo_s[:, :],
                                 preferred_element_type=jnp.float32)

        for st in range(LOG2_N):
            partner = my ^ (1 << st)
            rdma = pltpu.make_async_remote_copy(
                src_ref=out_ref,
                dst_ref=recv_ref.at[st],
                send_sem=send_sems.at[st],
                recv_sem=recv_sems.at[st],
                device_id=(partner,),
                device_id_type=pl.DeviceIdType.MESH,
            )
            rdma.start()
            rdma.wait()
            out_ref[:] = out_ref[:] + recv_ref[st]

    return pl.pallas_call(
        body,
        out_shape=jax.ShapeDtypeStruct((B, Sq, Dout), jnp.float32),
        in_specs=[
            pl.BlockSpec(memory_space=pltpu.VMEM),
            pl.BlockSpec(memory_space=pl.ANY),
            pl.BlockSpec(memory_space=pltpu.VMEM),
            pl.BlockSpec(memory_space=pltpu.VMEM),
            pl.BlockSpec(memory_space=pl.ANY),
        ],
        out_specs=pl.BlockSpec(memory_space=pltpu.VMEM),
        scratch_shapes=[
            pltpu.VMEM((D, Dq), jnp.float32),
            pltpu.VMEM((Dq, Dout), jnp.float32),
            pltpu.VMEM((LOG2_N, B, Sq, Dout), jnp.float32),
            pltpu.SemaphoreType.DMA((2,)),
            pltpu.SemaphoreType.DMA((LOG2_N,)),
            pltpu.SemaphoreType.DMA((LOG2_N,)),
        ],
    )(x, Wq, K_ext, V_ext, Wo)


# device time: 48540 ns/iter; 1.5797x vs baseline; 1.5797x over previous
import jax
import jax.numpy as jnp
from jax import lax
from jax.experimental import pallas as pl
from jax.experimental.pallas import tpu as pltpu

N_DEV = 32
LOG2_N = 5
RPB = 8


def kernel(x, Wq, K_ext, V_ext, Wo):
    B, Sq, D = x.shape
    _, Skv, Hl, Dh = K_ext.shape
    Dq = Hl * Dh
    Dout = Wo.shape[1]
    R = B * Sq

    def body(x_ref, wq_hbm, k_ref, v_ref, wo_hbm, out_ref,
             acc, wq_s, wo_s, rs_bufs, copy_sems,
             rs_send, rs_recv, ag_send, ag_recv):
        my = lax.axis_index("i")

        barrier = pltpu.get_barrier_semaphore()
        for k in range(LOG2_N):
            pl.semaphore_signal(
                barrier, inc=1, device_id=(my ^ (1 << k),),
                device_id_type=pl.DeviceIdType.MESH)
        pl.semaphore_wait(barrier, LOG2_N)

        cq = pltpu.make_async_copy(
            wq_hbm.at[:, pl.ds(my * Dq, Dq)], wq_s, copy_sems.at[0])
        co = pltpu.make_async_copy(
            wo_hbm.at[pl.ds(my * Dq, Dq), :], wo_s, copy_sems.at[1])
        cq.start()
        co.start()
        cq.wait()
        co.wait()

        for b in range(B):
            qb = jnp.dot(x_ref[b], wq_s[:, :],
                         preferred_element_type=jnp.float32)
            ctxs = []
            for h in range(Hl):
                q = qb[:, h * Dh:(h + 1) * Dh]
                kk = k_ref[b, :, h, :]
                vv = v_ref[b, :, h, :]
                s = lax.dot_general(
                    q, kk, (((1,), (1,)), ((), ())),
                    preferred_element_type=jnp.float32) * 0.125
                m = jnp.max(s, axis=1, keepdims=True)
                w = jnp.exp(s - m)
                w = w / jnp.sum(w, axis=1, keepdims=True)
                ctxs.append(jnp.dot(w, vv, preferred_element_type=jnp.float32))
            ctx = jnp.concatenate(ctxs, axis=1)
            acc[pl.ds(b * Sq, Sq), :] = jnp.dot(
                ctx, wo_s[:, :], preferred_element_type=jnp.float32)

        s_blk = my - my
        for k in range(LOG2_N):
            half = 1 << (4 - k)
            rows = half * RPB
            bit = (my >> k) & 1
            keep_start = s_blk + bit * half
            sent_start = s_blk + (1 - bit) * half
            rdma = pltpu.make_async_remote_copy(
                src_ref=acc.at[pl.ds(sent_start * RPB, rows), :],
                dst_ref=rs_bufs[k],
                send_sem=rs_send.at[k],
                recv_sem=rs_recv.at[k],
                device_id=(my ^ (1 << k),),
                device_id_type=pl.DeviceIdType.MESH,
            )
            rdma.start()
            rdma.wait()
            acc[pl.ds(keep_start * RPB, rows), :] = (
                acc[pl.ds(keep_start * RPB, rows), :] + rs_bufs[k][:, :])
            s_blk = keep_start

        for k in range(LOG2_N - 1, -1, -1):
            half = 1 << (4 - k)
            rows = half * RPB
            bit = (my >> k) & 1
            own_start = s_blk
            rdma = pltpu.make_async_remote_copy(
                src_ref=acc.at[pl.ds(own_start * RPB, rows), :],
                dst_ref=acc.at[pl.ds(own_start * RPB, rows), :],
                send_sem=ag_send.at[k],
                recv_sem=ag_recv.at[k],
                device_id=(my ^ (1 << k),),
                device_id_type=pl.DeviceIdType.MESH,
            )
            rdma.start()
            rdma.wait()
            s_blk = s_blk - bit * half

        out_ref[0] = acc[pl.ds(0, Sq), :]
        out_ref[1] = acc[pl.ds(Sq, Sq), :]

    rs_buf_shapes = [
        pltpu.VMEM(((1 << (4 - k)) * RPB, Dout), jnp.float32)
        for k in range(LOG2_N)
    ]

    return pl.pallas_call(
        body,
        out_shape=jax.ShapeDtypeStruct((B, Sq, Dout), jnp.float32),
        in_specs=[
            pl.BlockSpec(memory_space=pltpu.VMEM),
            pl.BlockSpec(memory_space=pl.ANY),
            pl.BlockSpec(memory_space=pltpu.VMEM),
            pl.BlockSpec(memory_space=pltpu.VMEM),
            pl.BlockSpec(memory_space=pl.ANY),
        ],
        out_specs=pl.BlockSpec(memory_space=pltpu.VMEM),
        scratch_shapes=[
            pltpu.VMEM((R, Dout), jnp.float32),
            pltpu.VMEM((D, Dq), jnp.float32),
            pltpu.VMEM((Dq, Dout), jnp.float32),
            rs_buf_shapes,
            pltpu.SemaphoreType.DMA((2,)),
            pltpu.SemaphoreType.DMA((LOG2_N,)),
            pltpu.SemaphoreType.DMA((LOG2_N,)),
            pltpu.SemaphoreType.DMA((LOG2_N,)),
            pltpu.SemaphoreType.DMA((LOG2_N,)),
        ],
        compiler_params=pltpu.CompilerParams(collective_id=0),
    )(x, Wq, K_ext, V_ext, Wo)


# device time: 38584 ns/iter; 1.9874x vs baseline; 1.2580x over previous
import jax
import jax.numpy as jnp
from jax import lax
from jax.experimental import pallas as pl
from jax.experimental.pallas import tpu as pltpu

N_DEV = 32
RPB = 8


def kernel(x, Wq, K_ext, V_ext, Wo):
    B, Sq, D = x.shape
    _, Skv, Hl, Dh = K_ext.shape
    Dq = Hl * Dh
    Dout = Wo.shape[1]
    R = B * Sq

    def body(x_ref, wq_hbm, k_ref, v_ref, wo_hbm, out_ref,
             acc, wq_s, wo_s, rs_buf, copy_sems,
             rs_send, rs_recv, ag_send, ag_recv):
        my = lax.axis_index("i")

        barrier = pltpu.get_barrier_semaphore()
        for r in range(1, N_DEV):
            pl.semaphore_signal(
                barrier, inc=1, device_id=((my + r) % N_DEV,),
                device_id_type=pl.DeviceIdType.MESH)
        pl.semaphore_wait(barrier, N_DEV - 1)

        cq = pltpu.make_async_copy(
            wq_hbm.at[:, pl.ds(my * Dq, Dq)], wq_s, copy_sems.at[0])
        co = pltpu.make_async_copy(
            wo_hbm.at[pl.ds(my * Dq, Dq), :], wo_s, copy_sems.at[1])
        cq.start()
        co.start()
        cq.wait()
        co.wait()

        for b in range(B):
            qb = jnp.dot(x_ref[b], wq_s[:, :],
                         preferred_element_type=jnp.float32)
            ctxs = []
            for h in range(Hl):
                q = qb[:, h * Dh:(h + 1) * Dh]
                kk = k_ref[b, :, h, :]
                vv = v_ref[b, :, h, :]
                s = lax.dot_general(
                    q, kk, (((1,), (1,)), ((), ())),
                    preferred_element_type=jnp.float32) * 0.125
                m = jnp.max(s, axis=1, keepdims=True)
                w = jnp.exp(s - m)
                w = w / jnp.sum(w, axis=1, keepdims=True)
                ctxs.append(jnp.dot(w, vv, preferred_element_type=jnp.float32))
            ctx = jnp.concatenate(ctxs, axis=1)
            acc[pl.ds(b * Sq, Sq), :] = jnp.dot(
                ctx, wo_s[:, :], preferred_element_type=jnp.float32)

        for r in range(1, N_DEV):
            peer = (my + r) % N_DEV
            rdma = pltpu.make_async_remote_copy(
                src_ref=acc.at[pl.ds(peer * RPB, RPB), :],
                dst_ref=rs_buf.at[my],
                send_sem=rs_send.at[r],
                recv_sem=rs_recv.at[my],
                device_id=(peer,),
                device_id_type=pl.DeviceIdType.MESH,
            )
            rdma.start()
        rs_buf[pl.ds(my, 1)] = acc[pl.ds(my * RPB, RPB), :].reshape(
            1, RPB, Dout)
        for r in range(1, N_DEV):
            peer = (my + r) % N_DEV
            pltpu.make_async_remote_copy(
                src_ref=rs_buf.at[peer],
                dst_ref=rs_buf.at[peer],
                send_sem=rs_send.at[r],
                recv_sem=rs_recv.at[peer],
                device_id=(peer,),
                device_id_type=pl.DeviceIdType.MESH,
            ).wait_recv()
        red = jnp.sum(rs_buf[:, :, :], axis=0)
        acc[pl.ds(my * RPB, RPB), :] = red

        for r in range(1, N_DEV):
            peer = (my + r) % N_DEV
            rdma = pltpu.make_async_remote_copy(
                src_ref=acc.at[pl.ds(my * RPB, RPB), :],
                dst_ref=acc.at[pl.ds(my * RPB, RPB), :],
                send_sem=ag_send.at[r],
                recv_sem=ag_recv.at[my],
                device_id=(peer,),
                device_id_type=pl.DeviceIdType.MESH,
            )
            rdma.start()
        for r in range(1, N_DEV):
            peer = (my + r) % N_DEV
            pltpu.make_async_remote_copy(
                src_ref=acc.at[pl.ds(peer * RPB, RPB), :],
                dst_ref=acc.at[pl.ds(peer * RPB, RPB), :],
                send_sem=ag_send.at[r],
                recv_sem=ag_recv.at[peer],
                device_id=(peer,),
                device_id_type=pl.DeviceIdType.MESH,
            ).wait_recv()

        for r in range(1, N_DEV):
            peer = (my + r) % N_DEV
            pltpu.make_async_remote_copy(
                src_ref=acc.at[pl.ds(peer * RPB, RPB), :],
                dst_ref=acc.at[pl.ds(peer * RPB, RPB), :],
                send_sem=rs_send.at[r],
                recv_sem=rs_recv.at[peer],
                device_id=(peer,),
                device_id_type=pl.DeviceIdType.MESH,
            ).wait_send()
            pltpu.make_async_remote_copy(
                src_ref=acc.at[pl.ds(my * RPB, RPB), :],
                dst_ref=acc.at[pl.ds(my * RPB, RPB), :],
                send_sem=ag_send.at[r],
                recv_sem=ag_recv.at[peer],
                device_id=(peer,),
                device_id_type=pl.DeviceIdType.MESH,
            ).wait_send()

        out_ref[0] = acc[pl.ds(0, Sq), :]
        out_ref[1] = acc[pl.ds(Sq, Sq), :]

    return pl.pallas_call(
        body,
        out_shape=jax.ShapeDtypeStruct((B, Sq, Dout), jnp.float32),
        in_specs=[
            pl.BlockSpec(memory_space=pltpu.VMEM),
            pl.BlockSpec(memory_space=pl.ANY),
            pl.BlockSpec(memory_space=pltpu.VMEM),
            pl.BlockSpec(memory_space=pltpu.VMEM),
            pl.BlockSpec(memory_space=pl.ANY),
        ],
        out_specs=pl.BlockSpec(memory_space=pltpu.VMEM),
        scratch_shapes=[
            pltpu.VMEM((R, Dout), jnp.float32),
            pltpu.VMEM((D, Dq), jnp.float32),
            pltpu.VMEM((Dq, Dout), jnp.float32),
            pltpu.VMEM((N_DEV, RPB, Dout), jnp.float32),
            pltpu.SemaphoreType.DMA((2,)),
            pltpu.SemaphoreType.DMA((N_DEV,)),
            pltpu.SemaphoreType.DMA((N_DEV,)),
            pltpu.SemaphoreType.DMA((N_DEV,)),
            pltpu.SemaphoreType.DMA((N_DEV,)),
        ],
        compiler_params=pltpu.CompilerParams(collective_id=0),
    )(x, Wq, K_ext, V_ext, Wo)
